# baseline (speedup 1.0000x reference)
import functools

import jax
import jax.numpy as jnp
from jax import lax
from jax.experimental import pallas as pl
from jax.experimental.pallas import tpu as pltpu
from jax.experimental.pallas import tpu_sc as plsc

ROWS = 16384
FEATS = 26
EMB = 64
B = ROWS * FEATS
NC, NS = 2, 16
NW = NC * NS
B_PER_W = B // NW
NBUF = 4
CHUNK = 416
NCHUNK = B_PER_W // CHUNK
NROUND = NCHUNK // NBUF

_mesh = plsc.VectorSubcoreMesh(core_axis_name="c", subcore_axis_name="s")


@functools.partial(
    pl.kernel,
    mesh=_mesh,
    out_type=jax.ShapeDtypeStruct((B, EMB), jnp.float32),
    compiler_params=pltpu.CompilerParams(use_tc_tiling_on_sc=False),
    scratch_types=[
        pltpu.VMEM((B_PER_W,), jnp.int32),
        [pltpu.VMEM((CHUNK, EMB), jnp.float32)] * NBUF,
        [pltpu.SemaphoreType.DMA] * NBUF,
        [pltpu.SemaphoreType.DMA] * NBUF,
    ],
)
def _gather_all(idx_hbm, table_hbm, out_hbm, idx_v, bufs, gsems, ssems):
    wid = lax.axis_index("s") * NC + lax.axis_index("c")
    base = wid * B_PER_W
    pltpu.sync_copy(idx_hbm.at[pl.ds(base, B_PER_W)], idx_v)

    def start_gather(off, buf, sem):
        pltpu.async_copy(table_hbm.at[idx_v.at[pl.ds(off, CHUNK)]], buf, sem)

    for b in range(2):
        start_gather(b * CHUNK, bufs[b], gsems[b])

    def ring_round(r, carry):
        for b in range(NBUF):
            g = r * NBUF + b
            off = g * CHUNK
            pltpu.make_async_copy(
                table_hbm.at[idx_v.at[pl.ds(off, CHUNK)]],
                bufs[b], gsems[b]).wait()
            pltpu.async_copy(bufs[b], out_hbm.at[pl.ds(base + off, CHUNK)],
                             ssems[b])
            bn = (b + 2) % NBUF

            @pl.when(g >= 2)
            def _():
                pltpu.make_async_copy(
                    bufs[bn],
                    out_hbm.at[pl.ds(base + (g - 2) * CHUNK, CHUNK)],
                    ssems[bn]).wait()

            @pl.when(g + 2 < NCHUNK)
            def _():
                start_gather((g + 2) * CHUNK, bufs[bn], gsems[bn])
        return carry

    lax.fori_loop(0, NROUND, ring_round, 0)

    for g in (NCHUNK - 2, NCHUNK - 1):
        b = g % NBUF
        pltpu.make_async_copy(
            bufs[b], out_hbm.at[pl.ds(base + g * CHUNK, CHUNK)],
            ssems[b]).wait()


def kernel(node_label_index, W):
    idx = node_label_index.reshape(-1).astype(jnp.int32)
    out = _gather_all(idx, W)
    return out.reshape(ROWS, FEATS, EMB)


# speedup vs baseline: 1.3154x; 1.3154x over previous
import functools

import jax
import jax.numpy as jnp
from jax import lax
from jax.experimental import pallas as pl
from jax.experimental.pallas import tpu as pltpu
from jax.experimental.pallas import tpu_sc as plsc

ROWS = 16384
FEATS = 26
EMB = 64
B = ROWS * FEATS
V = 1000000
NC, NS = 2, 16
NW = NC * NS
LANE = 16
PITCH = 2 * EMB

_CP = pltpu.CompilerParams(needs_layout_passes=False, use_tc_tiling_on_sc=True)
_mesh = plsc.VectorSubcoreMesh(core_axis_name="c", subcore_axis_name="s")

RBLK = 128
NFULL = V // RBLK
VTAIL = NFULL * RBLK
RREM = V - VTAIL
TPW = (NFULL + NW - 1) // NW


@functools.partial(
    pl.kernel,
    mesh=_mesh,
    out_type=jax.ShapeDtypeStruct((V, PITCH), jnp.float32),
    compiler_params=_CP,
    scratch_types=[
        [pltpu.VMEM((EMB, RBLK), jnp.float32)] * 2,
        [pltpu.VMEM((RBLK, PITCH), jnp.float32)] * 2,
        pltpu.VMEM((RREM * EMB,), jnp.float32),
        [pltpu.SemaphoreType.DMA] * 2,
        [pltpu.SemaphoreType.DMA] * 2,
    ],
)
def _reformat_table(wt_hbm, wtail_hbm, tbl_hbm, bufa, bufb, bufr, lsem, ssem):
    wid = lax.axis_index("s") * NC + lax.axis_index("c")
    iota = lax.iota(jnp.int32, LANE)

    def start_load(t, slot):
        blk = t * NW + wid

        @pl.when(blk < NFULL)
        def _():
            pltpu.async_copy(
                wt_hbm.at[:, pl.ds(blk * RBLK, RBLK)], bufa[slot], lsem[slot])

    start_load(0, 0)

    def proc(t, s):
        blk = t * NW + wid
        start_load(t + 1, 1 - s)

        @pl.when(blk < NFULL)
        def _():
            pltpu.make_async_copy(
                wt_hbm.at[:, pl.ds(blk * RBLK, RBLK)], bufa[s], lsem[s]).wait()
            @pl.when(t >= 2)
            def _():
                pltpu.make_async_copy(
                    bufb[s], tbl_hbm.at[pl.ds((blk - 2 * NW) * RBLK, RBLK)],
                    ssem[s]).wait()
            efull = jnp.full((LANE,), EMB, jnp.int32)

            def tloop(t, carry):
                for u in range(4):
                    cvec = lax.rem(iota + (t * 4 + u), efull)
                    for k in range(RBLK // LANE):
                        rvec = iota + (k * LANE)
                        v = plsc.load_gather(bufa[s], [cvec, rvec])
                        plsc.store_scatter(bufb[s], [rvec, cvec], v)
                return carry

            lax.fori_loop(0, EMB // 4, tloop, 0)
            pltpu.async_copy(
                bufb[s], tbl_hbm.at[pl.ds(blk * RBLK, RBLK)], ssem[s])

    def pair_step(q, carry):
        proc(2 * q, 0)
        proc(2 * q + 1, 1)
        return carry

    lax.fori_loop(0, TPW // 2, pair_step, 0)
    proc(TPW - 1, 0)

    last1 = (TPW - 2) * NW + wid
    pltpu.make_async_copy(
        bufb[1], tbl_hbm.at[pl.ds(last1 * RBLK, RBLK)], ssem[1]).wait()
    last0 = lax.select((TPW - 1) * NW + wid < NFULL,
                       (TPW - 1) * NW + wid, (TPW - 3) * NW + wid)
    pltpu.make_async_copy(
        bufb[0], tbl_hbm.at[pl.ds(last0 * RBLK, RBLK)], ssem[0]).wait()

    @pl.when(wid == 0)
    def _():
        pltpu.sync_copy(wtail_hbm, bufr)
        for j in range(RREM):
            jvec = jnp.full((LANE,), j, jnp.int32)
            for k in range(EMB // LANE):
                v = bufr[pl.ds(j * EMB + k * LANE, LANE)]
                plsc.store_scatter(bufb[0], [jvec, iota + (k * LANE)], v)
        pltpu.sync_copy(bufb[0].at[pl.ds(0, RREM)],
                        tbl_hbm.at[pl.ds(VTAIL, RREM)])


I_PER_W = ROWS // NW
IBLK = 128
QN = I_PER_W // IBLK
NUNIT = FEATS * QN
NUPAIR = NUNIT // 2


@functools.partial(
    pl.kernel,
    mesh=_mesh,
    out_type=jax.ShapeDtypeStruct((FEATS, EMB, ROWS), jnp.float32),
    compiler_params=_CP,
    scratch_types=[
        pltpu.VMEM((FEATS * I_PER_W,), jnp.int32),
        [pltpu.VMEM((IBLK, PITCH), jnp.float32)] * 2,
        [pltpu.VMEM((1, EMB, IBLK), jnp.float32)] * 2,
        pltpu.SemaphoreType.DMA,
        [pltpu.SemaphoreType.DMA] * 2,
        [pltpu.SemaphoreType.DMA] * 2,
    ],
)
def _gather_all(idxt_hbm, tbl_hbm, out_hbm, idx_v, bufg, buft, isem,
                gsems, ssems):
    wid = lax.axis_index("s") * NC + lax.axis_index("c")
    ibase = wid * I_PER_W
    iota = lax.iota(jnp.int32, LANE)
    efull = jnp.full((LANE,), EMB, jnp.int32)

    for j in range(FEATS):
        pltpu.async_copy(idxt_hbm.at[j, pl.ds(ibase, I_PER_W)],
                         idx_v.at[pl.ds(j * I_PER_W, I_PER_W)], isem)
    for j in range(FEATS):
        pltpu.make_async_copy(idxt_hbm.at[j, pl.ds(ibase, I_PER_W)],
                              idx_v.at[pl.ds(j * I_PER_W, I_PER_W)],
                              isem).wait()

    def start_gather(u, slot):
        @pl.when(u < NUNIT)
        def _():
            off = (u >> 2) * I_PER_W + (u & 3) * IBLK
            pltpu.async_copy(tbl_hbm.at[idx_v.at[pl.ds(off, IBLK)]],
                             bufg[slot], gsems[slot])

    start_gather(0, 0)

    def store_dst(u):
        return out_hbm.at[pl.ds(u >> 2, 1), :,
                          pl.ds(ibase + (u & 3) * IBLK, IBLK)]

    def proc(u, s):
        start_gather(u + 1, 1 - s)
        off = (u >> 2) * I_PER_W + (u & 3) * IBLK
        pltpu.make_async_copy(tbl_hbm.at[idx_v.at[pl.ds(off, IBLK)]],
                              bufg[s], gsems[s]).wait()

        @pl.when(u >= 2)
        def _():
            pltpu.make_async_copy(buft[s], store_dst(u - 2), ssems[s]).wait()

        zvec = iota * 0

        def tloop(t, carry):
            for v4 in range(4):
                cvec = lax.rem(iota + (t * 4 + v4), efull)
                for k in range(IBLK // LANE):
                    rvec = iota + (k * LANE)
                    v = plsc.load_gather(bufg[s], [rvec, cvec])
                    plsc.store_scatter(buft[s], [zvec, cvec, rvec], v)
            return carry

        lax.fori_loop(0, EMB // 4, tloop, 0)
        pltpu.async_copy(buft[s], store_dst(u), ssems[s])

    def pair(p, carry):
        proc(2 * p, 0)
        proc(2 * p + 1, 1)
        return carry

    lax.fori_loop(0, NUPAIR, pair, 0)
    pltpu.make_async_copy(buft[0], store_dst(NUNIT - 2), ssems[0]).wait()
    pltpu.make_async_copy(buft[1], store_dst(NUNIT - 1), ssems[1]).wait()


def kernel(node_label_index, W):
    idxt = node_label_index.T.astype(jnp.int32)
    wtail = W[VTAIL:, :].reshape(-1)
    tbl = _reformat_table(W.T, wtail)
    out_t = _gather_all(idxt, tbl)
    return jnp.transpose(out_t, (2, 0, 1))
